# baseline (speedup 1.0000x reference)
import functools

import jax
import jax.numpy as jnp
from jax import lax
from jax.experimental import pallas as pl
from jax.experimental.pallas import tpu as pltpu
from jax.experimental.pallas import tpu_sc as plsc

NC, NS = 2, 16
NW = NC * NS
V, D = 1000000, 32
B, T = 4096, 200
NTG, NBG = 25, 32
B_UNITS = 25


def _mesh():
    return plsc.VectorSubcoreMesh(
        core_axis_name="c", subcore_axis_name="s",
        num_cores=NC, num_subcores=NS,
    )


@jax.jit
def _run(idx, table):
    idx5 = (idx.T.reshape(NTG, 8, NBG, 128)
            .transpose(0, 2, 1, 3).reshape(NTG, NBG, 1024))

    @functools.partial(
        pl.kernel,
        mesh=_mesh(),
        out_type=jax.ShapeDtypeStruct((NTG, 8, 4, NBG, 8, 128), jnp.float32),
        scratch_types=[
            pltpu.VMEM((1024,), jnp.int32),
            pltpu.VMEM((1024, D), jnp.float32),
            pltpu.VMEM((8, 4, 8, 128), jnp.float32),
            pltpu.SemaphoreType.DMA,
        ],
        compiler_params=pltpu.CompilerParams(
            use_tc_tiling_on_sc=False, needs_layout_passes=False),
    )
    def kb(idx_hbm, table_hbm, out_hbm, idxv, rows, obuf, sem):
        wid = lax.axis_index("s") * NC + lax.axis_index("c")

        @pl.loop(0, B_UNITS)
        def _(u):
            q = wid + u * NW
            tg = q // NBG
            bg = q % NBG
            pltpu.sync_copy(idx_hbm.at[tg, bg], idxv)
            pltpu.async_copy(table_hbm.at[idxv], rows, sem).wait()

            @pl.loop(0, 8)
            def _(r):
                for tr in range(4):
                    for rr in range(8):
                        c = 8 * tr + rr
                        cvec = jnp.full((16,), c, jnp.int32)
                        for m in range(8):
                            jvec = jnp.full((16,), r * 128 + 16 * m,
                                            jnp.int32) + lax.iota(jnp.int32, 16)
                            vals = plsc.load_gather(rows, [jvec, cvec])
                            obuf[r, tr, rr, pl.ds(16 * m, 16)] = vals
                for tr in range(4):
                    pltpu.sync_copy(obuf.at[r, tr], out_hbm.at[tg, r, tr, bg])

    tbl = lax.optimization_barrier(table.reshape(V // 4, 128)).reshape(V, D)
    out5 = kb(idx5, tbl)
    out = (out5.transpose(3, 5, 0, 1, 2, 4).reshape(B, T, D))
    return out


def kernel(idx, token_embedding_table):
    return _run(idx.astype(jnp.int32), token_embedding_table)


# speedup vs baseline: 1.1004x; 1.1004x over previous
import functools

import jax
import jax.numpy as jnp
from jax import lax
from jax.experimental import pallas as pl
from jax.experimental.pallas import tpu as pltpu
from jax.experimental.pallas import tpu_sc as plsc

NC, NS = 2, 16
NW = NC * NS
V, D = 1000000, 32
B, T = 4096, 200
NTG, NBG = 25, 32
B_UNITS = 25


def _mesh():
    return plsc.VectorSubcoreMesh(
        core_axis_name="c", subcore_axis_name="s",
        num_cores=NC, num_subcores=NS,
    )


@jax.jit
def _run(idx, table):
    idx5 = (idx.T.reshape(NTG, 8, NBG, 128)
            .transpose(0, 2, 1, 3).reshape(NTG, NBG, 1024))

    @functools.partial(
        pl.kernel,
        mesh=_mesh(),
        out_type=jax.ShapeDtypeStruct((NTG, 8, 4, NBG, 8, 128), jnp.float32),
        scratch_types=[
            pltpu.VMEM((1024,), jnp.int32),
            pltpu.VMEM((1024,), jnp.int32),
            pltpu.VMEM((1024, D), jnp.float32),
            pltpu.VMEM((1024, D), jnp.float32),
            pltpu.VMEM((8, 4, 8, 128), jnp.float32),
            pltpu.SemaphoreType.DMA,
            pltpu.SemaphoreType.DMA,
            pltpu.SemaphoreType.DMA,
        ],
        compiler_params=pltpu.CompilerParams(
            use_tc_tiling_on_sc=False, needs_layout_passes=False),
    )
    def kb(idx_hbm, table_hbm, out_hbm,
           idxv0, idxv1, rows0, rows1, obuf, gsem0, gsem1, osem):
        wid = lax.axis_index("s") * NC + lax.axis_index("c")
        idxvs = (idxv0, idxv1)
        rowss = (rows0, rows1)
        gsems = (gsem0, gsem1)
        dummy = table_hbm.at[pl.ds(0, 1024)]

        def fetch(u, b):
            q = wid + u * NW
            pltpu.sync_copy(idx_hbm.at[q // NBG, q % NBG], idxvs[b])
            pltpu.async_copy(table_hbm.at[idxvs[b]], rowss[b], gsems[b])

        def process(u, b):
            q = wid + u * NW
            tg = q // NBG
            bg = q % NBG
            pltpu.make_async_copy(dummy, rowss[b], gsems[b]).wait()

            @pl.loop(0, 8)
            def _(r):
                for tr in range(4):
                    for rr in range(8):
                        c = 8 * tr + rr
                        cvec = jnp.full((16,), c, jnp.int32)
                        for m in range(8):
                            jvec = jnp.full((16,), r * 128 + 16 * m,
                                            jnp.int32) + lax.iota(jnp.int32, 16)
                            vals = plsc.load_gather(rowss[b], [jvec, cvec])
                            obuf[r, tr, rr, pl.ds(16 * m, 16)] = vals
                for tr in range(4):
                    pltpu.async_copy(obuf.at[r, tr], out_hbm.at[tg, r, tr, bg],
                                     osem)
            pltpu.make_async_copy(dummy, rowss[b], osem).wait()

        fetch(0, 0)

        @pl.loop(0, B_UNITS - 1, step=2)
        def _(g):
            fetch(g + 1, 1)
            process(g, 0)

            @pl.when(g + 2 < B_UNITS)
            def _():
                fetch(g + 2, 0)

            process(g + 1, 1)

        process(B_UNITS - 1, 0)

    out5 = kb(idx5, table)
    out = (out5.transpose(3, 5, 0, 1, 2, 4).reshape(B, T, D))
    return out


def kernel(idx, token_embedding_table):
    return _run(idx.astype(jnp.int32), token_embedding_table)
